# baseline (speedup 1.0000x reference)
import functools

import jax
import jax.numpy as jnp
from jax import lax
from jax.experimental import pallas as pl
from jax.experimental.pallas import tpu as pltpu
from jax.experimental.pallas import tpu_sc as plsc

D = 64
NNEG = 5
NC = 2
NS = 16
L = 16
NW = NC * NS
NV = D // L
NT = 1 + NNEG
PW = 128
D2 = 2 * D


@functools.lru_cache(maxsize=None)
def _make_sc_partials(B: int):
    assert B % NW == 0
    bpw = B // NW
    ng = bpw // L
    mesh = plsc.VectorSubcoreMesh(core_axis_name="c", subcore_axis_name="s")

    @functools.partial(
        pl.kernel,
        out_type=jax.ShapeDtypeStruct((NW, PW), jnp.float32),
        mesh=mesh,
        scratch_types=[
            pltpu.VMEM((NT + 1, bpw), jnp.int32),
            pltpu.VMEM((NT + 1, bpw), jnp.int32),
            pltpu.VMEM((NT + 1, bpw), jnp.int32),
            pltpu.VMEM((bpw, D2), jnp.float32),
            pltpu.VMEM((bpw, D2), jnp.float32),
            pltpu.VMEM((NNEG, bpw, D2), jnp.float32),
            pltpu.VMEM((PW,), jnp.float32),
            pltpu.SemaphoreType.DMA,
        ],
    )
    def sc_partials(iw_hbm, cw_hbm, nw_hbm, u_hbm, v_hbm, out_hbm,
                    idx_vm, idx_h, idx_p, u_rows, v_rows, n_rows, pout, sem):
        wid = lax.axis_index("s") * NC + lax.axis_index("c")
        base = wid * bpw

        pltpu.sync_copy(iw_hbm.at[pl.ds(base, bpw)], idx_vm.at[0])
        pltpu.sync_copy(cw_hbm.at[pl.ds(base, bpw)], idx_vm.at[1])
        for n in range(NNEG):
            pltpu.sync_copy(nw_hbm.at[pl.ds(n * B + base, bpw)],
                            idx_vm.at[2 + n])

        def split(g, carry):
            for t in range(NT + 1):
                v = idx_vm[t, pl.ds(g * L, L)]
                idx_h[t, pl.ds(g * L, L)] = v >> 1
                idx_p[t, pl.ds(g * L, L)] = (v & 1) * D
            return carry

        lax.fori_loop(0, ng, split, 0)

        copies = [
            pltpu.async_copy(u_hbm.at[idx_h.at[0]], u_rows, sem),
            pltpu.async_copy(v_hbm.at[idx_h.at[1]], v_rows, sem),
        ]
        for n in range(NNEG):
            copies.append(
                pltpu.async_copy(v_hbm.at[idx_h.at[2 + n]], n_rows.at[n],
                                 sem))
        for cp in copies:
            cp.wait()

        def body(g, accs):
            accs = list(accs)
            pu = idx_p[0, pl.ds(g * L, L)]
            pc = idx_p[1, pl.ds(g * L, L)]
            pn = [idx_p[2 + n, pl.ds(g * L, L)] for n in range(NNEG)]
            for k in range(L):
                i = g * L + k
                ou = pu[k]
                u = [u_rows[i, pl.ds(ou + j * L, L)] for j in range(NV)]
                oc = pc[k]
                for j in range(NV):
                    accs[j] = accs[j] + u[j] * v_rows[i, pl.ds(oc + j * L, L)]
                for n in range(NNEG):
                    on = pn[n][k]
                    for j in range(NV):
                        idx = (n + 1) * NV + j
                        accs[idx] = accs[idx] + (
                            u[j] * n_rows[n, i, pl.ds(on + j * L, L)])
            return tuple(accs)

        zero = jnp.zeros((L,), jnp.float32)
        accs = lax.fori_loop(0, ng, body, (zero,) * (NT * NV))
        for t in range(NT):
            a = accs[t * NV:(t + 1) * NV]
            pout[pl.ds(t * L, L)] = (a[0] + a[1]) + (a[2] + a[3])
        for t in range(NT, PW // L):
            pout[pl.ds(t * L, L)] = zero
        pltpu.sync_copy(pout, out_hbm.at[wid])

    return sc_partials


def _tc_combine_body(p_ref, o_ref):
    x = p_ref[...]
    grp = lax.broadcasted_iota(jnp.int32, x.shape, 1) // L

    def logsig(z):
        return jnp.minimum(z, 0.0) - jnp.log1p(jnp.exp(-jnp.abs(z)))

    s = [jnp.sum(jnp.where(grp == t, x, 0.0)) for t in range(NT)]
    loss = -(logsig(s[0]) + sum(logsig(-s[t]) for t in range(1, NT)))
    o_ref[...] = loss * jnp.ones((1, 1), jnp.float32)


@jax.jit
def kernel(input_words, context_words, neg_words, u_emb, v_emb):
    B = input_words.shape[0]
    V = u_emb.shape[0]
    partials = _make_sc_partials(B)(
        input_words.astype(jnp.int32),
        context_words.astype(jnp.int32),
        neg_words.astype(jnp.int32).reshape(-1),
        u_emb.reshape(V // 2, D2),
        v_emb.reshape(V // 2, D2),
    )
    loss = pl.pallas_call(
        _tc_combine_body,
        out_shape=jax.ShapeDtypeStruct((1, 1), jnp.float32),
    )(partials)
    return loss[0, 0]


# speedup vs baseline: 1.0110x; 1.0110x over previous
import functools

import jax
import jax.numpy as jnp
from jax import lax
from jax.experimental import pallas as pl
from jax.experimental.pallas import tpu as pltpu
from jax.experimental.pallas import tpu_sc as plsc

D = 64
NNEG = 5
NC = 2
NS = 16
L = 16
NW = NC * NS
NV = D // L
NT = 1 + NNEG
PW = 128
D2 = 2 * D
TCH = 2048


def _tc_transpose_body(in_ref, out_ref):
    x = in_ref[...]
    xt = x.T
    a = xt.reshape(TCH // 2, 2, D)
    out_ref[...] = jnp.concatenate([a[:, 0, :], a[:, 1, :]], axis=-1)


@functools.lru_cache(maxsize=None)
def _make_tc_transpose(V: int):
    grid = (V + TCH - 1) // TCH

    def run(t):
        return pl.pallas_call(
            _tc_transpose_body,
            grid=(grid,),
            in_specs=[pl.BlockSpec((D, TCH), lambda i: (0, i))],
            out_specs=pl.BlockSpec((TCH // 2, PW), lambda i: (i, 0)),
            out_shape=jax.ShapeDtypeStruct((V // 2, D2), jnp.float32),
        )(t)

    return run


@functools.lru_cache(maxsize=None)
def _make_sc_partials(B: int):
    assert B % NW == 0
    bpw = B // NW
    ng = bpw // L
    mesh = plsc.VectorSubcoreMesh(core_axis_name="c", subcore_axis_name="s")

    @functools.partial(
        pl.kernel,
        out_type=jax.ShapeDtypeStruct((NW, PW), jnp.float32),
        mesh=mesh,
        scratch_types=[
            pltpu.VMEM((NT + 1, bpw), jnp.int32),
            pltpu.VMEM((NT + 1, bpw), jnp.int32),
            pltpu.VMEM((NT + 1, bpw), jnp.int32),
            pltpu.VMEM((bpw, D2), jnp.float32),
            pltpu.VMEM((bpw, D2), jnp.float32),
            pltpu.VMEM((NNEG, bpw, D2), jnp.float32),
            pltpu.VMEM((PW,), jnp.float32),
            pltpu.SemaphoreType.DMA,
        ],
    )
    def sc_partials(iw_hbm, cw_hbm, nw_hbm, u_hbm, v_hbm, out_hbm,
                    idx_vm, idx_h, idx_p, u_rows, v_rows, n_rows, pout, sem):
        wid = lax.axis_index("s") * NC + lax.axis_index("c")
        base = wid * bpw

        pltpu.sync_copy(iw_hbm.at[pl.ds(base, bpw)], idx_vm.at[0])
        pltpu.sync_copy(cw_hbm.at[pl.ds(base, bpw)], idx_vm.at[1])
        for n in range(NNEG):
            pltpu.sync_copy(nw_hbm.at[pl.ds(n * B + base, bpw)],
                            idx_vm.at[2 + n])

        def split(g, carry):
            for t in range(NT + 1):
                v = idx_vm[t, pl.ds(g * L, L)]
                idx_h[t, pl.ds(g * L, L)] = v >> 1
                idx_p[t, pl.ds(g * L, L)] = (v & 1) * D
            return carry

        lax.fori_loop(0, ng, split, 0)

        copies = [
            pltpu.async_copy(u_hbm.at[idx_h.at[0]], u_rows, sem),
            pltpu.async_copy(v_hbm.at[idx_h.at[1]], v_rows, sem),
        ]
        for n in range(NNEG):
            copies.append(
                pltpu.async_copy(v_hbm.at[idx_h.at[2 + n]], n_rows.at[n],
                                 sem))
        for cp in copies:
            cp.wait()

        def body(g, accs):
            accs = list(accs)
            pu = idx_p[0, pl.ds(g * L, L)]
            pc = idx_p[1, pl.ds(g * L, L)]
            pn = [idx_p[2 + n, pl.ds(g * L, L)] for n in range(NNEG)]
            for k in range(L):
                i = g * L + k
                ou = pu[k]
                u = [u_rows[i, pl.ds(ou + j * L, L)] for j in range(NV)]
                oc = pc[k]
                for j in range(NV):
                    accs[j] = accs[j] + u[j] * v_rows[i, pl.ds(oc + j * L, L)]
                for n in range(NNEG):
                    on = pn[n][k]
                    for j in range(NV):
                        idx = (n + 1) * NV + j
                        accs[idx] = accs[idx] + (
                            u[j] * n_rows[n, i, pl.ds(on + j * L, L)])
            return tuple(accs)

        zero = jnp.zeros((L,), jnp.float32)
        accs = lax.fori_loop(0, ng, body, (zero,) * (NT * NV))
        for t in range(NT):
            a = accs[t * NV:(t + 1) * NV]
            pout[pl.ds(t * L, L)] = (a[0] + a[1]) + (a[2] + a[3])
        for t in range(NT, PW // L):
            pout[pl.ds(t * L, L)] = zero
        pltpu.sync_copy(pout, out_hbm.at[wid])

    return sc_partials


def _tc_combine_body(p_ref, o_ref):
    x = p_ref[...]
    grp = lax.broadcasted_iota(jnp.int32, x.shape, 1) // L

    def logsig(z):
        return jnp.minimum(z, 0.0) - jnp.log1p(jnp.exp(-jnp.abs(z)))

    s = [jnp.sum(jnp.where(grp == t, x, 0.0)) for t in range(NT)]
    loss = -(logsig(s[0]) + sum(logsig(-s[t]) for t in range(1, NT)))
    o_ref[...] = loss * jnp.ones((1, 1), jnp.float32)


@jax.jit
def kernel(input_words, context_words, neg_words, u_emb, v_emb):
    B = input_words.shape[0]
    V = u_emb.shape[0]
    tr = _make_tc_transpose(V)
    u2 = tr(u_emb.T)
    v2 = tr(v_emb.T)
    partials = _make_sc_partials(B)(
        input_words.astype(jnp.int32),
        context_words.astype(jnp.int32),
        neg_words.astype(jnp.int32).reshape(-1),
        u2,
        v2,
    )
    loss = pl.pallas_call(
        _tc_combine_body,
        out_shape=jax.ShapeDtypeStruct((1, 1), jnp.float32),
    )(partials)
    return loss[0, 0]


# speedup vs baseline: 1.2506x; 1.2370x over previous
import functools

import jax
import jax.numpy as jnp
from jax import lax
from jax.experimental import pallas as pl
from jax.experimental.pallas import tpu as pltpu
from jax.experimental.pallas import tpu_sc as plsc

D = 64
NNEG = 5
NC = 2
NS = 16
L = 16
NW = NC * NS
NV = D // L
NT = 1 + NNEG
PW = 128


@functools.lru_cache(maxsize=None)
def _make_sc_vgather(B: int):
    assert B % NW == 0
    bpw = B // NW
    mesh = plsc.VectorSubcoreMesh(core_axis_name="c", subcore_axis_name="s")

    @functools.partial(
        pl.kernel,
        out_type=jax.ShapeDtypeStruct((NT * B, D), jnp.float32),
        mesh=mesh,
        scratch_types=[
            pltpu.VMEM((NT, bpw), jnp.int32),
            pltpu.VMEM((NT, bpw, D), jnp.float32),
            pltpu.SemaphoreType.DMA,
        ],
    )
    def sc_vgather(cw_hbm, nw_hbm, v_hbm, out_hbm, idx_vm, rows, sem):
        wid = lax.axis_index("s") * NC + lax.axis_index("c")
        base = wid * bpw

        pltpu.sync_copy(cw_hbm.at[pl.ds(base, bpw)], idx_vm.at[0])
        for n in range(NNEG):
            pltpu.sync_copy(nw_hbm.at[pl.ds(n * B + base, bpw)],
                            idx_vm.at[1 + n])

        def fire(g, carry):
            vs = [idx_vm[t, pl.ds(g * L, L)] for t in range(NT)]
            for k in range(L):
                i = g * L + k
                for t in range(NT):
                    pltpu.async_copy(
                        v_hbm.at[pl.ds(vs[t][k], 1)],
                        rows.at[t].at[pl.ds(i, 1)], sem)
            return carry

        lax.fori_loop(0, bpw // L, fire, 0)
        for t in range(NT):
            pltpu.make_async_copy(v_hbm.at[pl.ds(0, bpw)], rows.at[t],
                                  sem).wait()
        for t in range(NT):
            pltpu.sync_copy(rows.at[t], out_hbm.at[pl.ds(t * B + base, bpw)])

    return sc_vgather


@functools.lru_cache(maxsize=None)
def _make_sc_udots(B: int):
    assert B % NW == 0
    bpw = B // NW
    mesh = plsc.VectorSubcoreMesh(core_axis_name="c", subcore_axis_name="s")

    @functools.partial(
        pl.kernel,
        out_type=jax.ShapeDtypeStruct((NW, PW), jnp.float32),
        mesh=mesh,
        compiler_params=pltpu.CompilerParams(use_tc_tiling_on_sc=False),
        scratch_types=[
            pltpu.VMEM((bpw,), jnp.int32),
            pltpu.VMEM((bpw, D), jnp.float32),
            pltpu.VMEM((NT, bpw, D), jnp.float32),
            pltpu.VMEM((PW,), jnp.float32),
            pltpu.SemaphoreType.DMA,
        ],
    )
    def sc_udots(iw_hbm, u_hbm, g_hbm, out_hbm,
                 idx_u, u_rows, v_rows, pout, sem):
        wid = lax.axis_index("s") * NC + lax.axis_index("c")
        base = wid * bpw

        pltpu.sync_copy(iw_hbm.at[pl.ds(base, bpw)], idx_u)
        cp = pltpu.async_copy(u_hbm.at[idx_u], u_rows, sem)
        for t in range(NT):
            pltpu.sync_copy(g_hbm.at[pl.ds(t * B + base, bpw)],
                            v_rows.at[t])
        cp.wait()

        def body(i, accs):
            accs = list(accs)
            u = [u_rows[i, pl.ds(j * L, L)] for j in range(NV)]
            for t in range(NT):
                for j in range(NV):
                    k = t * NV + j
                    accs[k] = accs[k] + u[j] * v_rows[t, i, pl.ds(j * L, L)]
            return tuple(accs)

        zero = jnp.zeros((L,), jnp.float32)
        accs = lax.fori_loop(0, bpw, body, (zero,) * (NT * NV))
        for t in range(NT):
            a = accs[t * NV:(t + 1) * NV]
            pout[pl.ds(t * L, L)] = (a[0] + a[1]) + (a[2] + a[3])
        for t in range(NT, PW // L):
            pout[pl.ds(t * L, L)] = zero
        pltpu.sync_copy(pout, out_hbm.at[wid])

    return sc_udots


def _tc_combine_body(p_ref, o_ref):
    x = p_ref[...]
    grp = lax.broadcasted_iota(jnp.int32, x.shape, 1) // L

    def logsig(z):
        return jnp.minimum(z, 0.0) - jnp.log1p(jnp.exp(-jnp.abs(z)))

    s = [jnp.sum(jnp.where(grp == t, x, 0.0)) for t in range(NT)]
    loss = -(logsig(s[0]) + sum(logsig(-s[t]) for t in range(1, NT)))
    o_ref[...] = loss * jnp.ones((1, 1), jnp.float32)


@jax.jit
def kernel(input_words, context_words, neg_words, u_emb, v_emb):
    B = input_words.shape[0]
    g = _make_sc_vgather(B)(
        context_words.astype(jnp.int32),
        neg_words.astype(jnp.int32).reshape(-1),
        v_emb,
    )
    partials = _make_sc_udots(B)(
        input_words.astype(jnp.int32),
        u_emb,
        g,
    )
    loss = pl.pallas_call(
        _tc_combine_body,
        out_shape=jax.ShapeDtypeStruct((1, 1), jnp.float32),
    )(partials)
    return loss[0, 0]


# speedup vs baseline: 2.8616x; 2.2882x over previous
import functools

import jax
import jax.numpy as jnp
from jax import lax
from jax.experimental import pallas as pl
from jax.experimental.pallas import tpu as pltpu
from jax.experimental.pallas import tpu_sc as plsc

D = 64
NNEG = 5
NC = 2
NS = 16
L = 16
NW = NC * NS
NV = D // L
NT = 1 + NNEG
NG = D // 8
PW = 128


@functools.lru_cache(maxsize=None)
def _make_sc_ufetch(B: int):
    assert B % NW == 0
    bpw = B // NW
    mesh = plsc.VectorSubcoreMesh(core_axis_name="c", subcore_axis_name="s")

    @functools.partial(
        pl.kernel,
        out_type=jax.ShapeDtypeStruct((B, D), jnp.float32),
        mesh=mesh,
        compiler_params=pltpu.CompilerParams(needs_layout_passes=False),
        scratch_types=[
            pltpu.VMEM((bpw,), jnp.int32),
            pltpu.VMEM((8 * NG, 8, 128), jnp.float32),
            pltpu.VMEM((bpw, D), jnp.float32),
            pltpu.SemaphoreType.DMA,
        ],
    )
    def sc_ufetch(iw_hbm, ut_hbm, out_hbm, idx_vm, ebuf, urows, sem):
        wid = lax.axis_index("s") * NC + lax.axis_index("c")
        base = wid * bpw

        pltpu.sync_copy(iw_hbm.at[pl.ds(base, bpw)], idx_vm)

        lane = lax.iota(jnp.int32, L)
        bl_a = lane >> 3
        bl_s = lane & 7

        def gbody(g, carry):
            vec = idx_vm[pl.ds(g * L, L)]
            for h in range(2):
                for e in range(8):
                    r = vec[h * 8 + e]
                    co = pl.multiple_of((r >> 7) * 128, 128)
                    for a in range(NG):
                        pltpu.async_copy(
                            ut_hbm.at[pl.ds(8 * a, 8), pl.ds(co, 128)],
                            ebuf.at[e * NG + a], sem)
                for e in range(8):
                    for a in range(NG):
                        pltpu.make_async_copy(
                            ut_hbm.at[pl.ds(0, 8), pl.ds(0, 128)],
                            ebuf.at[e * NG + a], sem).wait()
                for e in range(8):
                    r = vec[h * 8 + e]
                    c = jnp.full((L,), r & 127, jnp.int32)
                    i = g * L + h * 8 + e
                    for j in range(NV):
                        d0 = bl_a + (e * NG + 2 * j)
                        urows[i, pl.ds(j * L, L)] = plsc.load_gather(
                            ebuf, [d0, bl_s, c])
            return carry

        lax.fori_loop(0, bpw // L, gbody, 0)
        pltpu.sync_copy(urows, out_hbm.at[pl.ds(base, bpw)])

    return sc_ufetch


@functools.lru_cache(maxsize=None)
def _make_sc_vdots(B: int):
    assert B % NW == 0
    bpw = B // NW
    mesh = plsc.VectorSubcoreMesh(core_axis_name="c", subcore_axis_name="s")

    @functools.partial(
        pl.kernel,
        out_type=jax.ShapeDtypeStruct((NW, PW), jnp.float32),
        mesh=mesh,
        scratch_types=[
            pltpu.VMEM((NT, bpw), jnp.int32),
            pltpu.VMEM((bpw, D), jnp.float32),
            pltpu.VMEM((NT, bpw, D), jnp.float32),
            pltpu.VMEM((PW,), jnp.float32),
            pltpu.SemaphoreType.DMA,
        ],
    )
    def sc_vdots(cw_hbm, nw_hbm, v_hbm, gu_hbm, out_hbm,
                 idx_vm, u_rows, v_rows, pout, sem):
        wid = lax.axis_index("s") * NC + lax.axis_index("c")
        base = wid * bpw

        pltpu.sync_copy(cw_hbm.at[pl.ds(base, bpw)], idx_vm.at[0])
        for n in range(NNEG):
            pltpu.sync_copy(nw_hbm.at[pl.ds(n * B + base, bpw)],
                            idx_vm.at[1 + n])
        pltpu.sync_copy(gu_hbm.at[pl.ds(base, bpw)], u_rows)

        def fire(g, carry):
            vs = [idx_vm[t, pl.ds(g * L, L)] for t in range(NT)]
            for k in range(L):
                i = g * L + k
                for t in range(NT):
                    pltpu.async_copy(
                        v_hbm.at[pl.ds(vs[t][k], 1)],
                        v_rows.at[t].at[pl.ds(i, 1)], sem)
            return carry

        lax.fori_loop(0, bpw // L, fire, 0)
        for t in range(NT):
            pltpu.make_async_copy(v_hbm.at[pl.ds(0, bpw)], v_rows.at[t],
                                  sem).wait()

        def body(i, accs):
            accs = list(accs)
            u = [u_rows[i, pl.ds(j * L, L)] for j in range(NV)]
            for t in range(NT):
                for j in range(NV):
                    k = t * NV + j
                    accs[k] = accs[k] + u[j] * v_rows[t, i, pl.ds(j * L, L)]
            return tuple(accs)

        zero = jnp.zeros((L,), jnp.float32)
        accs = lax.fori_loop(0, bpw, body, (zero,) * (NT * NV))
        for t in range(NT):
            a = accs[t * NV:(t + 1) * NV]
            pout[pl.ds(t * L, L)] = (a[0] + a[1]) + (a[2] + a[3])
        for t in range(NT, PW // L):
            pout[pl.ds(t * L, L)] = zero
        pltpu.sync_copy(pout, out_hbm.at[wid])

    return sc_vdots


def _tc_combine_body(p_ref, o_ref):
    x = p_ref[...]
    grp = lax.broadcasted_iota(jnp.int32, x.shape, 1) // L

    def logsig(z):
        return jnp.minimum(z, 0.0) - jnp.log1p(jnp.exp(-jnp.abs(z)))

    s = [jnp.sum(jnp.where(grp == t, x, 0.0)) for t in range(NT)]
    loss = -(logsig(s[0]) + sum(logsig(-s[t]) for t in range(1, NT)))
    o_ref[...] = loss * jnp.ones((1, 1), jnp.float32)


@jax.jit
def kernel(input_words, context_words, neg_words, u_emb, v_emb):
    B = input_words.shape[0]
    gu = _make_sc_ufetch(B)(
        input_words.astype(jnp.int32),
        u_emb.T,
    )
    partials = _make_sc_vdots(B)(
        context_words.astype(jnp.int32),
        neg_words.astype(jnp.int32).reshape(-1),
        v_emb,
        gu,
    )
    loss = pl.pallas_call(
        _tc_combine_body,
        out_shape=jax.ShapeDtypeStruct((1, 1), jnp.float32),
    )(partials)
    return loss[0, 0]


# speedup vs baseline: 3.3698x; 1.1776x over previous
import functools

import jax
import jax.numpy as jnp
from jax import lax
from jax.experimental import pallas as pl
from jax.experimental.pallas import tpu as pltpu
from jax.experimental.pallas import tpu_sc as plsc

D = 64
NNEG = 5
NC = 2
NS = 16
L = 16
NW = NC * NS
NV = D // L
NT = 1 + NNEG
NSETS = NT + 1
PW = 128


@functools.lru_cache(maxsize=None)
def _make_sc_dots(B: int):
    assert B % NW == 0
    bpw = B // NW
    ng = bpw // L
    mesh = plsc.VectorSubcoreMesh(core_axis_name="c", subcore_axis_name="s")

    @functools.partial(
        pl.kernel,
        out_type=jax.ShapeDtypeStruct((NW, PW), jnp.float32),
        mesh=mesh,
        compiler_params=pltpu.CompilerParams(needs_layout_passes=False),
        scratch_types=[
            pltpu.VMEM((NSETS, bpw), jnp.int32),
            pltpu.VMEM((2 * NSETS, D, 128), jnp.float32),
            pltpu.VMEM((PW,), jnp.float32),
            pltpu.SemaphoreType.DMA,
        ],
    )
    def sc_dots(iw_hbm, cw_hbm, nw_hbm, ut_hbm, vt_hbm, out_hbm,
                idx_vm, ebuf, pout, sem):
        wid = lax.axis_index("s") * NC + lax.axis_index("c")
        base = wid * bpw

        pltpu.sync_copy(iw_hbm.at[pl.ds(base, bpw)], idx_vm.at[0])
        pltpu.sync_copy(cw_hbm.at[pl.ds(base, bpw)], idx_vm.at[1])
        for n in range(NNEG):
            pltpu.sync_copy(nw_hbm.at[pl.ds(n * B + base, bpw)],
                            idx_vm.at[2 + n])

        lane = lax.iota(jnp.int32, L)

        def tab(t):
            return ut_hbm if t == 0 else vt_hbm

        def fire(bank, rs):
            for t in range(NSETS):
                co = pl.multiple_of((rs[t] >> 7) * 128, 128)
                pltpu.async_copy(tab(t).at[:, pl.ds(co, 128)],
                                 ebuf.at[bank * NSETS + t], sem)

        vecs0 = [idx_vm[t, pl.ds(0, L)] for t in range(NSETS)]
        fire(0, [vecs0[t][0] for t in range(NSETS)])
        fire(1, [vecs0[t][1] for t in range(NSETS)])

        def gbody(g, accs):
            accs = list(accs)
            cvecs = [idx_vm[t, pl.ds(g * L, L)] for t in range(NSETS)]
            for k in range(L):
                bank = k & 1
                for t in range(NSETS):
                    pltpu.make_async_copy(tab(t).at[:, pl.ds(0, 128)],
                                          ebuf.at[bank * NSETS + t],
                                          sem).wait()
                cu = jnp.full((L,), cvecs[0][k] & 127, jnp.int32)
                u = [plsc.load_gather(ebuf.at[bank * NSETS],
                                      [j * L + lane, cu])
                     for j in range(NV)]
                for t in range(NT):
                    cv = jnp.full((L,), cvecs[1 + t][k] & 127, jnp.int32)
                    for j in range(NV):
                        w = plsc.load_gather(ebuf.at[bank * NSETS + 1 + t],
                                             [j * L + lane, cv])
                        accs[t * NV + j] = accs[t * NV + j] + u[j] * w
                if k < L - 2:
                    fire(bank, [cvecs[t][k + 2] for t in range(NSETS)])
                else:
                    @pl.when(g < ng - 1)
                    def _():
                        nvecs = [idx_vm[t, pl.ds((g + 1) * L, L)]
                                 for t in range(NSETS)]
                        fire(bank, [nvecs[t][k + 2 - L]
                                    for t in range(NSETS)])
            return tuple(accs)

        zero = jnp.zeros((L,), jnp.float32)
        accs = lax.fori_loop(0, ng, gbody, (zero,) * (NT * NV))
        for t in range(NT):
            a = accs[t * NV:(t + 1) * NV]
            pout[pl.ds(t * L, L)] = (a[0] + a[1]) + (a[2] + a[3])
        for t in range(NT, PW // L):
            pout[pl.ds(t * L, L)] = zero
        pltpu.sync_copy(pout, out_hbm.at[wid])

    return sc_dots


def _tc_combine_body(p_ref, o_ref):
    x = p_ref[...]
    grp = lax.broadcasted_iota(jnp.int32, x.shape, 1) // L

    def logsig(z):
        return jnp.minimum(z, 0.0) - jnp.log1p(jnp.exp(-jnp.abs(z)))

    s = [jnp.sum(jnp.where(grp == t, x, 0.0)) for t in range(NT)]
    loss = -(logsig(s[0]) + sum(logsig(-s[t]) for t in range(1, NT)))
    o_ref[...] = loss * jnp.ones((1, 1), jnp.float32)


@jax.jit
def kernel(input_words, context_words, neg_words, u_emb, v_emb):
    B = input_words.shape[0]
    partials = _make_sc_dots(B)(
        input_words.astype(jnp.int32),
        context_words.astype(jnp.int32),
        neg_words.astype(jnp.int32).reshape(-1),
        u_emb.T,
        v_emb.T,
    )
    loss = pl.pallas_call(
        _tc_combine_body,
        out_shape=jax.ShapeDtypeStruct((1, 1), jnp.float32),
    )(partials)
    return loss[0, 0]
